# baseline (speedup 1.0000x reference)
import functools

import jax
import jax.numpy as jnp
from jax import lax
from jax.experimental import pallas as pl
from jax.experimental.pallas import tpu as pltpu

_MAX_DISTANCE = 32
_NBUF = 6


def _stream_fn(
    tabs_ref,
    kt_hbm,
    vt_hbm,
    ko_hbm,
    vo_hbm,
    biast_ref,
    kbuf,
    vbuf,
    tt_ref,
    insem,
    outsem,
    *,
    h,
    hd,
    s,
):
    m = s * hd
    n_rows = 2 * _MAX_DISTANCE + 1
    n_cols = tabs_ref.shape[1]

    r = jax.lax.broadcasted_iota(jnp.int32, (m, n_rows), 1)
    mm = jax.lax.broadcasted_iota(jnp.int32, (m, n_rows), 0)
    y = mm // hd
    z = mm % hd
    idx = jnp.clip(y - z, -_MAX_DISTANCE, _MAX_DISTANCE) + _MAX_DISTANCE
    onehot = (r == idx).astype(jnp.float32)
    tt = jnp.dot(onehot, tabs_ref[...], preferred_element_type=jnp.float32)
    tt_ref[...] = tt
    biast_ref[...] = tt[:, 2 * h :]

    def start_in(u, slot):
        pltpu.make_async_copy(kt_hbm.at[u], kbuf.at[slot], insem.at[0, slot]).start()
        pltpu.make_async_copy(vt_hbm.at[u], vbuf.at[slot], insem.at[1, slot]).start()

    for b in range(_NBUF - 1):
        start_in(b, b)

    def body(u, carry):
        nxt = u + _NBUF - 1

        @pl.when(nxt < h)
        def _prefetch():
            slot_n = lax.rem(nxt, _NBUF)

            @pl.when(nxt >= _NBUF)
            def _wait_out_free():
                pltpu.make_async_copy(
                    kbuf.at[slot_n], ko_hbm.at[nxt - _NBUF], outsem.at[0, slot_n]
                ).wait()
                pltpu.make_async_copy(
                    vbuf.at[slot_n], vo_hbm.at[nxt - _NBUF], outsem.at[1, slot_n]
                ).wait()

            start_in(nxt, slot_n)

        slot = lax.rem(u, _NBUF)
        pltpu.make_async_copy(kt_hbm.at[u], kbuf.at[slot], insem.at[0, slot]).wait()
        pltpu.make_async_copy(vt_hbm.at[u], vbuf.at[slot], insem.at[1, slot]).wait()

        rr = jax.lax.broadcasted_iota(jnp.int32, (n_cols, 2), 0)
        cc = jax.lax.broadcasted_iota(jnp.int32, (n_cols, 2), 1)
        sel = ((rr == u) & (cc == 0)) | ((rr == u + h) & (cc == 1))
        cols = jnp.dot(
            tt_ref[...], sel.astype(jnp.float32), preferred_element_type=jnp.float32
        )

        kbuf[slot] = kbuf[slot] + cols[:, 0:1]
        vbuf[slot] = vbuf[slot] + cols[:, 1:2]
        pltpu.make_async_copy(kbuf.at[slot], ko_hbm.at[u], outsem.at[0, slot]).start()
        pltpu.make_async_copy(vbuf.at[slot], vo_hbm.at[u], outsem.at[1, slot]).start()
        return carry

    lax.fori_loop(0, h, body, 0)

    for j in range(_NBUF):
        u = h - _NBUF + j
        slot = u % _NBUF
        pltpu.make_async_copy(kbuf.at[slot], ko_hbm.at[u], outsem.at[0, slot]).wait()
        pltpu.make_async_copy(vbuf.at[slot], vo_hbm.at[u], outsem.at[1, slot]).wait()


@jax.jit
def kernel(q, k, v, bias_table, embed_k, embed_v):
    del q
    B, H, S, HD = k.shape
    M = S * HD
    N_ROWS = 2 * _MAX_DISTANCE + 1

    kt = k.transpose(1, 2, 3, 0).reshape(H, M, B)
    vt = v.transpose(1, 2, 3, 0).reshape(H, M, B)

    tabs = jnp.concatenate(
        [
            embed_k,
            embed_v,
            jnp.pad(bias_table, ((0, 1), (0, 0))),
        ],
        axis=1,
    )

    ko, vo, biast = pl.pallas_call(
        functools.partial(_stream_fn, h=H, hd=HD, s=S),
        in_specs=[
            pl.BlockSpec((N_ROWS, 3 * H), lambda: (0, 0)),
            pl.BlockSpec(memory_space=pl.ANY),
            pl.BlockSpec(memory_space=pl.ANY),
        ],
        out_specs=[
            pl.BlockSpec(memory_space=pl.ANY),
            pl.BlockSpec(memory_space=pl.ANY),
            pl.BlockSpec((M, H), lambda: (0, 0)),
        ],
        out_shape=[
            jax.ShapeDtypeStruct((H, M, B), jnp.float32),
            jax.ShapeDtypeStruct((H, M, B), jnp.float32),
            jax.ShapeDtypeStruct((M, H), jnp.float32),
        ],
        scratch_shapes=[
            pltpu.VMEM((_NBUF, M, B), jnp.float32),
            pltpu.VMEM((_NBUF, M, B), jnp.float32),
            pltpu.VMEM((M, 3 * H), jnp.float32),
            pltpu.SemaphoreType.DMA((2, _NBUF)),
            pltpu.SemaphoreType.DMA((2, _NBUF)),
        ],
    )(tabs, kt, vt)

    k_out = ko.reshape(H, S, HD, B).transpose(3, 0, 1, 2)
    v_out = vo.reshape(H, S, HD, B).transpose(3, 0, 1, 2)
    bias = biast.T.reshape(H, S, S)
    return (k_out, v_out, bias)


# speedup vs baseline: 1.0224x; 1.0224x over previous
import functools

import jax
import jax.numpy as jnp
from jax import lax
from jax.experimental import pallas as pl
from jax.experimental.pallas import tpu as pltpu

_MAX_DISTANCE = 32
_NBUF = 3


def _stream_fn(
    tabs_ref,
    kt_hbm,
    vt_hbm,
    ko_hbm,
    vo_hbm,
    biast_ref,
    kbuf,
    vbuf,
    kobuf,
    vobuf,
    tt_ref,
    insem,
    outsem,
    *,
    h,
    hd,
    s,
):
    m = s * hd
    n_rows = 2 * _MAX_DISTANCE + 1
    n_cols = tabs_ref.shape[1]

    r = jax.lax.broadcasted_iota(jnp.int32, (m, n_rows), 1)
    mm = jax.lax.broadcasted_iota(jnp.int32, (m, n_rows), 0)
    y = mm // hd
    z = mm % hd
    idx = jnp.clip(y - z, -_MAX_DISTANCE, _MAX_DISTANCE) + _MAX_DISTANCE
    onehot = (r == idx).astype(jnp.float32)
    tt = jnp.dot(onehot, tabs_ref[...], preferred_element_type=jnp.float32)
    tt_ref[...] = tt
    biast_ref[...] = tt[:, 2 * h :]

    def start_in(u, slot):
        pltpu.make_async_copy(kt_hbm.at[u], kbuf.at[slot], insem.at[0, slot]).start()
        pltpu.make_async_copy(vt_hbm.at[u], vbuf.at[slot], insem.at[1, slot]).start()

    for b in range(_NBUF - 1):
        start_in(b, b)

    def body(u, carry):
        nxt = u + _NBUF - 1

        @pl.when(nxt < h)
        def _prefetch():
            start_in(nxt, lax.rem(nxt, _NBUF))

        slot = lax.rem(u, _NBUF)
        pltpu.make_async_copy(kt_hbm.at[u], kbuf.at[slot], insem.at[0, slot]).wait()
        pltpu.make_async_copy(vt_hbm.at[u], vbuf.at[slot], insem.at[1, slot]).wait()

        rr = jax.lax.broadcasted_iota(jnp.int32, (n_cols, 2), 0)
        cc = jax.lax.broadcasted_iota(jnp.int32, (n_cols, 2), 1)
        sel = ((rr == u) & (cc == 0)) | ((rr == u + h) & (cc == 1))
        cols = jnp.dot(
            tt_ref[...], sel.astype(jnp.float32), preferred_element_type=jnp.float32
        )

        @pl.when(u >= _NBUF)
        def _wait_out_free():
            pltpu.make_async_copy(
                kobuf.at[slot], ko_hbm.at[u - _NBUF], outsem.at[0, slot]
            ).wait()
            pltpu.make_async_copy(
                vobuf.at[slot], vo_hbm.at[u - _NBUF], outsem.at[1, slot]
            ).wait()

        kobuf[slot] = kbuf[slot] + cols[:, 0:1]
        vobuf[slot] = vbuf[slot] + cols[:, 1:2]
        pltpu.make_async_copy(kobuf.at[slot], ko_hbm.at[u], outsem.at[0, slot]).start()
        pltpu.make_async_copy(vobuf.at[slot], vo_hbm.at[u], outsem.at[1, slot]).start()
        return carry

    lax.fori_loop(0, h, body, 0)

    for j in range(_NBUF):
        u = h - _NBUF + j
        slot = u % _NBUF
        pltpu.make_async_copy(kobuf.at[slot], ko_hbm.at[u], outsem.at[0, slot]).wait()
        pltpu.make_async_copy(vobuf.at[slot], vo_hbm.at[u], outsem.at[1, slot]).wait()


@jax.jit
def kernel(q, k, v, bias_table, embed_k, embed_v):
    del q
    B, H, S, HD = k.shape
    M = S * HD
    N_ROWS = 2 * _MAX_DISTANCE + 1

    kt = k.transpose(1, 2, 3, 0).reshape(H, M, B)
    vt = v.transpose(1, 2, 3, 0).reshape(H, M, B)

    tabs = jnp.concatenate(
        [
            embed_k,
            embed_v,
            jnp.pad(bias_table, ((0, 1), (0, 0))),
        ],
        axis=1,
    )

    ko, vo, biast = pl.pallas_call(
        functools.partial(_stream_fn, h=H, hd=HD, s=S),
        in_specs=[
            pl.BlockSpec((N_ROWS, 3 * H), lambda: (0, 0)),
            pl.BlockSpec(memory_space=pl.ANY),
            pl.BlockSpec(memory_space=pl.ANY),
        ],
        out_specs=[
            pl.BlockSpec(memory_space=pl.ANY),
            pl.BlockSpec(memory_space=pl.ANY),
            pl.BlockSpec((M, H), lambda: (0, 0)),
        ],
        out_shape=[
            jax.ShapeDtypeStruct((H, M, B), jnp.float32),
            jax.ShapeDtypeStruct((H, M, B), jnp.float32),
            jax.ShapeDtypeStruct((M, H), jnp.float32),
        ],
        scratch_shapes=[
            pltpu.VMEM((_NBUF, M, B), jnp.float32),
            pltpu.VMEM((_NBUF, M, B), jnp.float32),
            pltpu.VMEM((_NBUF, M, B), jnp.float32),
            pltpu.VMEM((_NBUF, M, B), jnp.float32),
            pltpu.VMEM((M, 3 * H), jnp.float32),
            pltpu.SemaphoreType.DMA((2, _NBUF)),
            pltpu.SemaphoreType.DMA((2, _NBUF)),
        ],
    )(tabs, kt, vt)

    k_out = ko.reshape(H, S, HD, B).transpose(3, 0, 1, 2)
    v_out = vo.reshape(H, S, HD, B).transpose(3, 0, 1, 2)
    bias = biast.T.reshape(H, S, S)
    return (k_out, v_out, bias)
